# baseline (speedup 1.0000x reference)
import jax
import jax.numpy as jnp
from jax.experimental import pallas as pl
from jax.experimental.pallas import tpu as pltpu
from jax.experimental.pallas import tpu_sc as plsc

NUM_CORES = 2
NUM_SUBCORES = 16
NUM_WORKERS = NUM_CORES * NUM_SUBCORES


def _sc_gather_both(user_ids, item_ids, user_table, item_table):
    B = user_ids.shape[0]
    H = user_table.shape[1]
    per_w = B // NUM_WORKERS
    mesh = plsc.VectorSubcoreMesh(core_axis_name="c", subcore_axis_name="s")

    @pl.kernel(
        out_type=(
            jax.ShapeDtypeStruct((B, H), user_table.dtype),
            jax.ShapeDtypeStruct((B, H), item_table.dtype),
        ),
        mesh=mesh,
        scratch_types=[
            pltpu.VMEM((per_w,), jnp.int32),
            pltpu.VMEM((per_w,), jnp.int32),
            pltpu.SemaphoreType.DMA,
        ],
    )
    def gather_kernel(ut_hbm, it_hbm, ui_hbm, ii_hbm, uo_hbm, io_hbm,
                      uidx_v, iidx_v, sem):
        wid = jax.lax.axis_index("s") * NUM_CORES + jax.lax.axis_index("c")
        base = wid * per_w
        pltpu.sync_copy(ui_hbm.at[pl.ds(base, per_w)], uidx_v)
        pltpu.sync_copy(ii_hbm.at[pl.ds(base, per_w)], iidx_v)

        @pl.loop(0, per_w, step=16)
        def _(g):
            uvec = uidx_v[pl.ds(g, 16)]
            ivec = iidx_v[pl.ds(g, 16)]
            for i in range(16):
                pltpu.async_copy(ut_hbm.at[uvec[i]], uo_hbm.at[base + g + i], sem)
                pltpu.async_copy(it_hbm.at[ivec[i]], io_hbm.at[base + g + i], sem)

        @pl.loop(0, 2 * per_w)
        def _(j):
            pltpu.make_async_copy(ut_hbm.at[0], uo_hbm.at[0], sem).wait()

    return gather_kernel(user_table, item_table, user_ids, item_ids)


def _mlp_body(ue_ref, ie_ref,
              u_W1, u_b1, u_W2, u_b2, u_W3, u_b3,
              i_W1, i_b1, i_W2, i_b2, i_W3, i_b3,
              p_W1, p_b1, p_W2, p_b2, out_ref):
    f32 = jnp.float32
    ue = ue_ref[...]
    ue = jnp.maximum(jnp.dot(ue, u_W1[...], preferred_element_type=f32) + u_b1[...], 0.0)
    ue = jnp.maximum(jnp.dot(ue, u_W2[...], preferred_element_type=f32) + u_b2[...], 0.0)
    ue = jnp.maximum(jnp.dot(ue, u_W3[...], preferred_element_type=f32) + u_b3[...], 0.0)
    ie = ie_ref[...]
    ie = jnp.maximum(jnp.dot(ie, i_W1[...], preferred_element_type=f32) + i_b1[...], 0.0)
    ie = jnp.maximum(jnp.dot(ie, i_W2[...], preferred_element_type=f32) + i_b2[...], 0.0)
    ie = jnp.maximum(jnp.dot(ie, i_W3[...], preferred_element_type=f32) + i_b3[...], 0.0)
    H = ue.shape[1]
    h = (jnp.dot(ue, p_W1[:H, :], preferred_element_type=f32)
         + jnp.dot(ie, p_W1[H:, :], preferred_element_type=f32) + p_b1[...])
    h = jnp.maximum(h, 0.0)
    out_ref[...] = jnp.dot(h, p_W2[...], preferred_element_type=f32) + p_b2[...]


def kernel(user_ids, item_ids, user_table, item_table,
           u_W1, u_b1, u_W2, u_b2, u_W3, u_b3,
           i_W1, i_b1, i_W2, i_b2, i_W3, i_b3,
           p_W1, p_b1, p_W2, p_b2):
    B = user_ids.shape[0]
    H = user_table.shape[1]
    ue, ie = _sc_gather_both(user_ids, item_ids, user_table, item_table)

    BLK = 2048
    full = lambda shape: pl.BlockSpec(shape, lambda i: tuple(0 for _ in shape))
    preds = pl.pallas_call(
        _mlp_body,
        grid=(B // BLK,),
        in_specs=[
            pl.BlockSpec((BLK, H), lambda i: (i, 0)),
            pl.BlockSpec((BLK, H), lambda i: (i, 0)),
            full(u_W1.shape), full(u_b1.shape), full(u_W2.shape), full(u_b2.shape),
            full(u_W3.shape), full(u_b3.shape),
            full(i_W1.shape), full(i_b1.shape), full(i_W2.shape), full(i_b2.shape),
            full(i_W3.shape), full(i_b3.shape),
            full(p_W1.shape), full(p_b1.shape), full(p_W2.shape), full(p_b2.shape),
        ],
        out_specs=pl.BlockSpec((BLK, 1), lambda i: (i, 0)),
        out_shape=jax.ShapeDtypeStruct((B, 1), jnp.float32),
    )(ue, ie,
      u_W1, u_b1, u_W2, u_b2, u_W3, u_b3,
      i_W1, i_b1, i_W2, i_b2, i_W3, i_b3,
      p_W1, p_b1, p_W2, p_b2)
    return preds.reshape(-1)


# speedup vs baseline: 1.0046x; 1.0046x over previous
import jax
import jax.numpy as jnp
from jax.experimental import pallas as pl
from jax.experimental.pallas import tpu as pltpu
from jax.experimental.pallas import tpu_sc as plsc

NUM_CORES = 2
NUM_SUBCORES = 16
NUM_WORKERS = NUM_CORES * NUM_SUBCORES
NSEM = 4


def _sc_gather_tiles(user_ids, item_ids, user_table, item_table):
    B = user_ids.shape[0]
    H = user_table.shape[1]
    n_tiles = user_table.shape[0] // 8
    per_w = B // NUM_WORKERS
    mesh = plsc.VectorSubcoreMesh(core_axis_name="c", subcore_axis_name="s")

    @pl.kernel(
        out_type=(
            jax.ShapeDtypeStruct((B, H), user_table.dtype),
            jax.ShapeDtypeStruct((B, H), item_table.dtype),
        ),
        mesh=mesh,
        scratch_types=[
            pltpu.VMEM((per_w,), jnp.int32),
            pltpu.VMEM((per_w,), jnp.int32),
            [pltpu.SemaphoreType.DMA] * NSEM,
        ],
    )
    def gather_kernel(ut_hbm, it_hbm, ui_hbm, ii_hbm, uo_hbm, io_hbm,
                      uidx_v, iidx_v, sems):
        wid = jax.lax.axis_index("s") * NUM_CORES + jax.lax.axis_index("c")
        base = wid * per_w
        pltpu.sync_copy(ui_hbm.at[pl.ds(base, per_w)], uidx_v)
        pltpu.sync_copy(ii_hbm.at[pl.ds(base, per_w)], iidx_v)

        @pl.loop(0, per_w, step=16)
        def _(g):
            uvec = uidx_v[pl.ds(g, 16)]
            ivec = iidx_v[pl.ds(g, 16)]
            for i in range(16):
                pltpu.async_copy(ut_hbm.at[uvec[i]], uo_hbm.at[base + g + i],
                                 sems[i % NSEM])
                pltpu.async_copy(it_hbm.at[ivec[i]], io_hbm.at[base + g + i],
                                 sems[(i + 1) % NSEM])

        @pl.loop(0, 2 * per_w // NSEM)
        def _(j):
            for k in range(NSEM):
                pltpu.make_async_copy(ut_hbm.at[0], uo_hbm.at[0], sems[k]).wait()

    return gather_kernel(user_table, item_table, user_ids, item_ids)


def _mlp_body(uew_ref, iew_ref,
              u_W1, u_b1, u_W2, u_b2, u_W3, u_b3,
              i_W1, i_b1, i_W2, i_b2, i_W3, i_b3,
              p_W1, p_b1, p_W2, p_b2, out_ref):
    f32 = jnp.float32
    bf16 = jnp.bfloat16
    H = u_W1.shape[0]

    def dense(x, W, b, relu=True):
        y = jnp.dot(x.astype(bf16), W[...].astype(bf16),
                    preferred_element_type=f32) + b[...]
        return jnp.maximum(y, 0.0) if relu else y

    ue = uew_ref[:, :H]
    ie = iew_ref[:, :H]
    ue = dense(dense(dense(ue, u_W1, u_b1), u_W2, u_b2), u_W3, u_b3)
    ie = dense(dense(dense(ie, i_W1, i_b1), i_W2, i_b2), i_W3, i_b3)
    h = (jnp.dot(ue.astype(bf16), p_W1[:H, :].astype(bf16), preferred_element_type=f32)
         + jnp.dot(ie.astype(bf16), p_W1[H:, :].astype(bf16), preferred_element_type=f32)
         + p_b1[...])
    h = jnp.maximum(h, 0.0)
    out_ref[...] = dense(h, p_W2, p_b2, relu=False)


def kernel(user_ids, item_ids, user_table, item_table,
           u_W1, u_b1, u_W2, u_b2, u_W3, u_b3,
           i_W1, i_b1, i_W2, i_b2, i_W3, i_b3,
           p_W1, p_b1, p_W2, p_b2):
    B = user_ids.shape[0]
    H = user_table.shape[1]
    uew, iew = _sc_gather_tiles(user_ids, item_ids, user_table, item_table)

    BLK = 2048
    full = lambda shape: pl.BlockSpec(shape, lambda i: tuple(0 for _ in shape))
    preds = pl.pallas_call(
        _mlp_body,
        grid=(B // BLK,),
        in_specs=[
            pl.BlockSpec((BLK, H), lambda i: (i, 0)),
            pl.BlockSpec((BLK, H), lambda i: (i, 0)),
            full(u_W1.shape), full(u_b1.shape), full(u_W2.shape), full(u_b2.shape),
            full(u_W3.shape), full(u_b3.shape),
            full(i_W1.shape), full(i_b1.shape), full(i_W2.shape), full(i_b2.shape),
            full(i_W3.shape), full(i_b3.shape),
            full(p_W1.shape), full(p_b1.shape), full(p_W2.shape), full(p_b2.shape),
        ],
        out_specs=pl.BlockSpec((BLK, 1), lambda i: (i, 0)),
        out_shape=jax.ShapeDtypeStruct((B, 1), jnp.float32),
    )(uew, iew,
      u_W1, u_b1, u_W2, u_b2, u_W3, u_b3,
      i_W1, i_b1, i_W2, i_b2, i_W3, i_b3,
      p_W1, p_b1, p_W2, p_b2)
    return preds.reshape(-1)


# speedup vs baseline: 1.0144x; 1.0097x over previous
import jax
import jax.numpy as jnp
from jax.experimental import pallas as pl
from jax.experimental.pallas import tpu as pltpu
from jax.experimental.pallas import tpu_sc as plsc

NUM_CORES = 2
NUM_SUBCORES = 16
NUM_WORKERS = NUM_CORES * NUM_SUBCORES
CHUNK = 128
PACK_BLK = 10000


def _pack_body(u_ref, i_ref, out_ref):
    out_ref[:, : u_ref.shape[1]] = u_ref[...]
    out_ref[:, u_ref.shape[1]:] = i_ref[...]


def _pack_tables(user_table, item_table):
    n, H = user_table.shape
    return pl.pallas_call(
        _pack_body,
        grid=(n // PACK_BLK,),
        in_specs=[
            pl.BlockSpec((PACK_BLK, H), lambda i: (i, 0)),
            pl.BlockSpec((PACK_BLK, H), lambda i: (i, 0)),
        ],
        out_specs=pl.BlockSpec((PACK_BLK, 2 * H), lambda i: (i, 0)),
        out_shape=jax.ShapeDtypeStruct((n, 2 * H), jnp.float32),
    )(user_table, item_table)


def _sc_gather(packed, ids):
    n, W = packed.shape
    NB = ids.shape[0]
    per_w = NB // NUM_WORKERS
    mesh = plsc.VectorSubcoreMesh(core_axis_name="c", subcore_axis_name="s")

    @pl.kernel(
        out_type=jax.ShapeDtypeStruct((NB, W), jnp.float32),
        mesh=mesh,
        scratch_types=[
            pltpu.VMEM((per_w,), jnp.int32),
            pltpu.VMEM((CHUNK, W), jnp.float32),
            pltpu.VMEM((CHUNK, W), jnp.float32),
            pltpu.SemaphoreType.DMA,
            pltpu.SemaphoreType.DMA,
        ],
    )
    def gather_kernel(tab_hbm, ids_hbm, out_hbm, idx_v, rows_a, rows_b, sem_a, sem_b):
        wid = jax.lax.axis_index("s") * NUM_CORES + jax.lax.axis_index("c")
        base = wid * per_w
        pltpu.sync_copy(ids_hbm.at[pl.ds(base, per_w)], idx_v)

        pltpu.async_copy(tab_hbm.at[idx_v.at[pl.ds(0, CHUNK)]], rows_a, sem_a).wait()

        @pl.loop(0, per_w // CHUNK // 2)
        def _(h):
            g = h * 2
            wr_a = pltpu.async_copy(rows_a, out_hbm.at[pl.ds(base + g * CHUNK, CHUNK)], sem_a)
            gt_b = pltpu.async_copy(
                tab_hbm.at[idx_v.at[pl.ds((g + 1) * CHUNK, CHUNK)]], rows_b, sem_b)
            wr_a.wait()
            gt_b.wait()
            wr_b = pltpu.async_copy(rows_b, out_hbm.at[pl.ds(base + (g + 1) * CHUNK, CHUNK)], sem_b)
            is_last = g + 2 >= per_w // CHUNK
            nxt = jnp.where(is_last, 0, (g + 2) * CHUNK)
            gt_a = pltpu.async_copy(tab_hbm.at[idx_v.at[pl.ds(nxt, CHUNK)]], rows_a, sem_a)
            wr_b.wait()
            gt_a.wait()

    return gather_kernel(packed, ids)


def _mlp_body(uew_ref, iew_ref,
              u_W1, u_b1, u_W2, u_b2, u_W3, u_b3,
              i_W1, i_b1, i_W2, i_b2, i_W3, i_b3,
              p_W1, p_b1, p_W2, p_b2, out_ref):
    f32 = jnp.float32
    bf16 = jnp.bfloat16
    H = u_W1.shape[0]

    def dense(x, W, b, relu=True):
        y = jnp.dot(x.astype(bf16), W[...].astype(bf16),
                    preferred_element_type=f32) + b[...]
        return jnp.maximum(y, 0.0) if relu else y

    ue = uew_ref[:, :H]
    ie = iew_ref[:, H:]
    ue = dense(dense(dense(ue, u_W1, u_b1), u_W2, u_b2), u_W3, u_b3)
    ie = dense(dense(dense(ie, i_W1, i_b1), i_W2, i_b2), i_W3, i_b3)
    h = (jnp.dot(ue.astype(bf16), p_W1[:H, :].astype(bf16), preferred_element_type=f32)
         + jnp.dot(ie.astype(bf16), p_W1[H:, :].astype(bf16), preferred_element_type=f32)
         + p_b1[...])
    h = jnp.maximum(h, 0.0)
    out_ref[...] = dense(h, p_W2, p_b2, relu=False)


def kernel(user_ids, item_ids, user_table, item_table,
           u_W1, u_b1, u_W2, u_b2, u_W3, u_b3,
           i_W1, i_b1, i_W2, i_b2, i_W3, i_b3,
           p_W1, p_b1, p_W2, p_b2):
    B = user_ids.shape[0]
    H = user_table.shape[1]
    packed = _pack_tables(user_table, item_table)
    ids = jnp.concatenate([user_ids, item_ids])
    rows = _sc_gather(packed, ids)

    BLK = 2048
    full = lambda shape: pl.BlockSpec(shape, lambda i: tuple(0 for _ in shape))
    nblk = B // BLK
    preds = pl.pallas_call(
        _mlp_body,
        grid=(nblk,),
        in_specs=[
            pl.BlockSpec((BLK, 2 * H), lambda i: (i, 0)),
            pl.BlockSpec((BLK, 2 * H), lambda i, _n=nblk: (i + _n, 0)),
            full(u_W1.shape), full(u_b1.shape), full(u_W2.shape), full(u_b2.shape),
            full(u_W3.shape), full(u_b3.shape),
            full(i_W1.shape), full(i_b1.shape), full(i_W2.shape), full(i_b2.shape),
            full(i_W3.shape), full(i_b3.shape),
            full(p_W1.shape), full(p_b1.shape), full(p_W2.shape), full(p_b2.shape),
        ],
        out_specs=pl.BlockSpec((BLK, 1), lambda i: (i, 0)),
        out_shape=jax.ShapeDtypeStruct((B, 1), jnp.float32),
    )(rows, rows,
      u_W1, u_b1, u_W2, u_b2, u_W3, u_b3,
      i_W1, i_b1, i_W2, i_b2, i_W3, i_b3,
      p_W1, p_b1, p_W2, p_b2)
    return preds.reshape(-1)
